# baseline (speedup 1.0000x reference)
import jax
import jax.numpy as jnp
from jax import lax
from jax.experimental import pallas as pl
from jax.experimental.pallas import tpu as pltpu
from jax.experimental.pallas import tpu_sc as plsc

NUM_BUCKETS = 32
NUM_HEADS = 16
Q = 4096
K = 4096
BQ = 128
BAND = 384
D2W = 640
DLOC = 768

_THRESH = (1, 2, 3, 4, 5, 6, 7, 8, 12, 16, 23, 32, 46, 64, 91)


def _sc_body(table_hbm, d2_hbm, table_v, dloc_v, rows_v, sem):
    wid = lax.axis_index("s") * 2 + lax.axis_index("c")
    h = wid // 2
    r0 = (wid % 2) * (BQ // 2)

    pltpu.sync_copy(table_hbm, table_v)
    iota = lax.iota(jnp.int32, 16)
    hvec = jnp.zeros((16,), jnp.int32) + h

    for c in range(DLOC // 16):
        k = iota + c * 16
        n = 384 - k
        an = jnp.abs(n)
        g = jnp.zeros((16,), jnp.int32)
        for t in _THRESH:
            g = g + (an >= t).astype(jnp.int32)
        bucket = jnp.where(n < 0, g + 16, g)
        dloc_v[pl.ds(c * 16, 16)] = plsc.load_gather(table_v, [bucket, hvec])

    def build_row(r, _):
        base = 128 - r0 - r
        for c in range(D2W // 16):
            idx = iota + c * 16 + base
            rows_v[pl.ds(r * D2W + c * 16, 16)] = plsc.load_gather(dloc_v, [idx])
        return 0

    lax.fori_loop(0, BQ // 2, build_row, 0, unroll=False)

    dst = pl.multiple_of((h * BQ + r0) * D2W, D2W)
    pltpu.async_copy(rows_v, d2_hbm.at[pl.ds(dst, (BQ // 2) * D2W)], sem).wait()


@jax.jit
def _sc_d2(table):
    flat = pl.kernel(
        _sc_body,
        out_type=jax.ShapeDtypeStruct((NUM_HEADS * BQ * D2W,), jnp.float32),
        mesh=plsc.VectorSubcoreMesh(core_axis_name="c", subcore_axis_name="s"),
        compiler_params=pltpu.CompilerParams(needs_layout_passes=False),
        scratch_types=[
            pltpu.VMEM((NUM_BUCKETS, NUM_HEADS), jnp.float32),
            pltpu.VMEM((DLOC,), jnp.float32),
            pltpu.VMEM(((BQ // 2) * D2W,), jnp.float32),
            pltpu.SemaphoreType.DMA,
        ],
    )(table)
    return flat.reshape(NUM_HEADS, BQ, D2W)


def _tc_body(d2_ref, out_ref):
    ib = pl.program_id(1)
    i0 = ib * BQ

    c_past = d2_ref[0, 0, 0]
    c_future = d2_ref[0, 0, D2W - 1]
    cs = pl.multiple_of(jnp.clip(i0 - BQ, 0, K - BAND), BQ)

    col = jax.lax.broadcasted_iota(jnp.int32, (BQ, K), 1)
    out_ref[0, 0] = jnp.where(col >= cs + 192, c_future, c_past)
    ms = pl.multiple_of(cs - i0 + 256, BQ)
    out_ref[0, 0, :, pl.ds(cs, BAND)] = d2_ref[0, :, pl.ds(ms, BAND)]


@jax.jit
def _tc_bias(d2):
    return pl.pallas_call(
        _tc_body,
        grid=(NUM_HEADS, Q // BQ),
        in_specs=[pl.BlockSpec((1, BQ, D2W), lambda h, ib: (h, 0, 0))],
        out_specs=pl.BlockSpec((1, 1, BQ, K), lambda h, ib: (0, h, ib, 0)),
        out_shape=jax.ShapeDtypeStruct((1, NUM_HEADS, Q, K), jnp.float32),
    )(d2)


def kernel(query_length, key_length, relative_attention_bias):
    del query_length, key_length
    d2 = _sc_d2(relative_attention_bias)
    return _tc_bias(d2)


# speedup vs baseline: 1.1332x; 1.1332x over previous
import jax
import jax.numpy as jnp
from jax import lax
from jax.experimental import pallas as pl
from jax.experimental.pallas import tpu as pltpu
from jax.experimental.pallas import tpu_sc as plsc

NUM_BUCKETS = 32
NUM_HEADS = 16
Q = 4096
K = 4096
BQ = 256
BAND = 512
D2W = 768
DLOC = 1024

_THRESH = (1, 2, 3, 4, 5, 6, 7, 8, 12, 16, 23, 32, 46, 64, 91)


def _sc_body(table_hbm, d2_hbm, table_v, dloc_v, rows_v, sem):
    wid = lax.axis_index("s") * 2 + lax.axis_index("c")
    h = wid // 2
    r0 = (wid % 2) * (BQ // 2)

    pltpu.sync_copy(table_hbm, table_v)
    iota = lax.iota(jnp.int32, 16)
    hvec = jnp.zeros((16,), jnp.int32) + h

    for c in range(DLOC // 16):
        k = iota + c * 16
        n = 512 - k
        an = jnp.abs(n)
        g = jnp.zeros((16,), jnp.int32)
        for t in _THRESH:
            g = g + (an >= t).astype(jnp.int32)
        bucket = jnp.where(n < 0, g + 16, g)
        dloc_v[pl.ds(c * 16, 16)] = plsc.load_gather(table_v, [bucket, hvec])

    def build_row(r, _):
        base = 256 - r0 - r
        for c in range(D2W // 16):
            idx = iota + c * 16 + base
            rows_v[pl.ds(r * D2W + c * 16, 16)] = plsc.load_gather(dloc_v, [idx])
        return 0

    lax.fori_loop(0, BQ // 2, build_row, 0, unroll=False)

    dst = pl.multiple_of((h * BQ + r0) * D2W, D2W)
    pltpu.async_copy(rows_v, d2_hbm.at[pl.ds(dst, (BQ // 2) * D2W)], sem).wait()


@jax.jit
def _sc_d2(table):
    flat = pl.kernel(
        _sc_body,
        out_type=jax.ShapeDtypeStruct((NUM_HEADS * BQ * D2W,), jnp.float32),
        mesh=plsc.VectorSubcoreMesh(core_axis_name="c", subcore_axis_name="s"),
        compiler_params=pltpu.CompilerParams(needs_layout_passes=False),
        scratch_types=[
            pltpu.VMEM((NUM_BUCKETS, NUM_HEADS), jnp.float32),
            pltpu.VMEM((DLOC,), jnp.float32),
            pltpu.VMEM(((BQ // 2) * D2W,), jnp.float32),
            pltpu.SemaphoreType.DMA,
        ],
    )(table)
    return flat.reshape(NUM_HEADS, BQ, D2W)


def _tc_body(d2_ref, out_ref):
    ib = pl.program_id(1)
    i0 = ib * BQ

    c_past = d2_ref[0, 0, 0]
    c_future = d2_ref[0, 0, D2W - 1]
    cs = pl.multiple_of(jnp.clip(i0 - 128, 0, K - BAND), 128)

    col = jax.lax.broadcasted_iota(jnp.int32, (BQ, K), 1)
    out_ref[0, 0] = jnp.where(col >= cs + 256, c_future, c_past)
    ms = pl.multiple_of(cs - i0 + 256, 128)
    out_ref[0, 0, :, pl.ds(cs, BAND)] = d2_ref[0, :, pl.ds(ms, BAND)]


@jax.jit
def _tc_bias(d2):
    return pl.pallas_call(
        _tc_body,
        grid=(NUM_HEADS, Q // BQ),
        in_specs=[pl.BlockSpec((1, BQ, D2W), lambda h, ib: (h, 0, 0))],
        out_specs=pl.BlockSpec((1, 1, BQ, K), lambda h, ib: (0, h, ib, 0)),
        out_shape=jax.ShapeDtypeStruct((1, NUM_HEADS, Q, K), jnp.float32),
    )(d2)


def kernel(query_length, key_length, relative_attention_bias):
    del query_length, key_length
    d2 = _sc_d2(relative_attention_bias)
    return _tc_bias(d2)


# speedup vs baseline: 1.1796x; 1.0410x over previous
import jax
import jax.numpy as jnp
from jax import lax
from jax.experimental import pallas as pl
from jax.experimental.pallas import tpu as pltpu
from jax.experimental.pallas import tpu_sc as plsc

NUM_BUCKETS = 32
NUM_HEADS = 16
Q = 4096
K = 4096
BQ = 256
BAND = 512
D2W = 768
DLOC = 1024

_THRESH = (1, 2, 3, 4, 5, 6, 7, 8, 12, 16, 23, 32, 46, 64, 91)


def _sc_body(table_hbm, d2_hbm, table_v, dloc_v, rows_v, sem):
    wid = lax.axis_index("s") * 2 + lax.axis_index("c")
    h = wid // 2
    r0 = (wid % 2) * (BQ // 2)

    pltpu.sync_copy(table_hbm, table_v)
    iota = lax.iota(jnp.int32, 16)
    hvec = jnp.zeros((16,), jnp.int32) + h

    for c in range(DLOC // 16):
        k = iota + c * 16
        n = 512 - k
        an = jnp.abs(n)
        g = jnp.zeros((16,), jnp.int32)
        for t in _THRESH:
            g = g + (an >= t).astype(jnp.int32)
        bucket = jnp.where(n < 0, g + 16, g)
        dloc_v[pl.ds(c * 16, 16)] = plsc.load_gather(table_v, [bucket, hvec])

    def build_row(r, _):
        base = 256 - r0 - r
        for c in range(D2W // 16):
            idx = iota + c * 16 + base
            rows_v[pl.ds(r * D2W + c * 16, 16)] = plsc.load_gather(dloc_v, [idx])
        return 0

    lax.fori_loop(0, BQ // 2, build_row, 0, unroll=False)

    dst = pl.multiple_of((h * BQ + r0) * D2W, D2W)
    pltpu.async_copy(rows_v, d2_hbm.at[pl.ds(dst, (BQ // 2) * D2W)], sem).wait()


@jax.jit
def _sc_d2(table):
    flat = pl.kernel(
        _sc_body,
        out_type=jax.ShapeDtypeStruct((NUM_HEADS * BQ * D2W,), jnp.float32),
        mesh=plsc.VectorSubcoreMesh(core_axis_name="c", subcore_axis_name="s"),
        compiler_params=pltpu.CompilerParams(needs_layout_passes=False),
        scratch_types=[
            pltpu.VMEM((NUM_BUCKETS, NUM_HEADS), jnp.float32),
            pltpu.VMEM((DLOC,), jnp.float32),
            pltpu.VMEM(((BQ // 2) * D2W,), jnp.float32),
            pltpu.SemaphoreType.DMA,
        ],
    )(table)
    return flat.reshape(NUM_HEADS, BQ, D2W)


TBQ = 512
NSUB = TBQ // BQ


def _tc_body(d2_ref, out_ref):
    ib = pl.program_id(1)
    i0 = ib * TBQ

    c_past = d2_ref[0, 0, 0]
    c_future = d2_ref[0, 0, D2W - 1]

    cs_last = pl.multiple_of(
        jnp.clip(i0 + (NSUB - 1) * BQ - 128, 0, K - BAND), 128)
    col = jax.lax.broadcasted_iota(jnp.int32, (TBQ, K), 1)
    out_ref[0, 0] = jnp.where(col >= cs_last + 128, c_future, c_past)

    for sub in range(NSUB):
        i0s = i0 + sub * BQ
        cs = pl.multiple_of(jnp.clip(i0s - 128, 0, K - BAND), 128)
        ms = pl.multiple_of(cs - i0s + 256, 128)
        out_ref[0, 0, sub * BQ:(sub + 1) * BQ, pl.ds(cs, BAND)] = (
            d2_ref[0, :, pl.ds(ms, BAND)])


@jax.jit
def _tc_bias(d2):
    return pl.pallas_call(
        _tc_body,
        grid=(NUM_HEADS, Q // TBQ),
        in_specs=[pl.BlockSpec((1, BQ, D2W), lambda h, ib: (h, 0, 0))],
        out_specs=pl.BlockSpec((1, 1, TBQ, K), lambda h, ib: (0, h, ib, 0)),
        out_shape=jax.ShapeDtypeStruct((1, NUM_HEADS, Q, K), jnp.float32),
    )(d2)


def kernel(query_length, key_length, relative_attention_bias):
    del query_length, key_length
    d2 = _sc_d2(relative_attention_bias)
    return _tc_bias(d2)


# speedup vs baseline: 1.1812x; 1.0014x over previous
import jax
import jax.numpy as jnp
from jax import lax
from jax.experimental import pallas as pl
from jax.experimental.pallas import tpu as pltpu
from jax.experimental.pallas import tpu_sc as plsc

NUM_BUCKETS = 32
NUM_HEADS = 16
Q = 4096
K = 4096
BQ = 256
BAND = 512
D2W = 768
DLOC = 1024

_THRESH = (1, 2, 3, 4, 5, 6, 7, 8, 12, 16, 23, 32, 46, 64, 91)


def _sc_body(table_hbm, d2_hbm, table_v, dloc_v, rows_v, sem):
    wid = lax.axis_index("s") * 2 + lax.axis_index("c")
    h = wid // 2
    r0 = (wid % 2) * (BQ // 2)

    pltpu.sync_copy(table_hbm, table_v)
    iota = lax.iota(jnp.int32, 16)
    hvec = jnp.zeros((16,), jnp.int32) + h

    for c in range(DLOC // 16):
        k = iota + c * 16
        n = 512 - k
        an = jnp.abs(n)
        g = jnp.zeros((16,), jnp.int32)
        for t in _THRESH:
            g = g + (an >= t).astype(jnp.int32)
        bucket = jnp.where(n < 0, g + 16, g)
        dloc_v[pl.ds(c * 16, 16)] = plsc.load_gather(table_v, [bucket, hvec])

    def build_row(r, _):
        base = 256 - r0 - r
        for c in range(D2W // 16):
            idx = iota + c * 16 + base
            rows_v[pl.ds(r * D2W + c * 16, 16)] = plsc.load_gather(dloc_v, [idx])
        return 0

    lax.fori_loop(0, BQ // 2, build_row, 0, unroll=False)

    dst = pl.multiple_of((h * BQ + r0) * D2W, D2W)
    pltpu.async_copy(rows_v, d2_hbm.at[pl.ds(dst, (BQ // 2) * D2W)], sem).wait()


@jax.jit
def _sc_d2(table):
    flat = pl.kernel(
        _sc_body,
        out_type=jax.ShapeDtypeStruct((NUM_HEADS * BQ * D2W,), jnp.float32),
        mesh=plsc.VectorSubcoreMesh(core_axis_name="c", subcore_axis_name="s"),
        compiler_params=pltpu.CompilerParams(needs_layout_passes=False),
        scratch_types=[
            pltpu.VMEM((NUM_BUCKETS, NUM_HEADS), jnp.float32),
            pltpu.VMEM((DLOC,), jnp.float32),
            pltpu.VMEM(((BQ // 2) * D2W,), jnp.float32),
            pltpu.SemaphoreType.DMA,
        ],
    )(table)
    return flat.reshape(NUM_HEADS, BQ, D2W)


TBQ = 1024
NSUB = TBQ // BQ


def _tc_body(d2_ref, out_ref):
    ib = pl.program_id(1)
    i0 = ib * TBQ

    c_past = d2_ref[0, 0, 0]
    c_future = d2_ref[0, 0, D2W - 1]

    row = jax.lax.broadcasted_iota(jnp.int32, (TBQ, 1), 0)
    rgrp = row - jax.lax.rem(row, BQ)
    split = jnp.clip(i0 + rgrp - 128, 0, K - BAND) + 128
    col = jax.lax.broadcasted_iota(jnp.int32, (TBQ, K), 1)
    out_ref[0, 0] = jnp.where(col >= split, c_future, c_past)

    for sub in range(NSUB):
        i0s = i0 + sub * BQ
        cs = pl.multiple_of(jnp.clip(i0s - 128, 0, K - BAND), 128)
        ms = pl.multiple_of(cs - i0s + 256, 128)
        out_ref[0, 0, sub * BQ:(sub + 1) * BQ, pl.ds(cs, BAND)] = (
            d2_ref[0, :, pl.ds(ms, BAND)])


@jax.jit
def _tc_bias(d2):
    return pl.pallas_call(
        _tc_body,
        grid=(NUM_HEADS, Q // TBQ),
        in_specs=[pl.BlockSpec((1, BQ, D2W), lambda h, ib: (h, 0, 0))],
        out_specs=pl.BlockSpec((1, 1, TBQ, K), lambda h, ib: (0, h, ib, 0)),
        out_shape=jax.ShapeDtypeStruct((1, NUM_HEADS, Q, K), jnp.float32),
    )(d2)


def kernel(query_length, key_length, relative_attention_bias):
    del query_length, key_length
    d2 = _sc_d2(relative_attention_bias)
    return _tc_bias(d2)


# speedup vs baseline: 1.2486x; 1.0570x over previous
import jax
import jax.numpy as jnp
from jax import lax
from jax.experimental import pallas as pl
from jax.experimental.pallas import tpu as pltpu
from jax.experimental.pallas import tpu_sc as plsc

NUM_BUCKETS = 32
NUM_HEADS = 16
Q = 4096
K = 4096
BQ = 128
BAND = 384
D2W = 640
DLOC = 768

_THRESH = (1, 2, 3, 4, 5, 6, 7, 8, 12, 16, 23, 32, 46, 64, 91)


def _sc_body(table_hbm, d2_hbm, table_v, dloc_v, rows_v, sem):
    wid = lax.axis_index("s") * 2 + lax.axis_index("c")
    h = wid // 2
    r0 = (wid % 2) * (BQ // 2)

    pltpu.sync_copy(table_hbm, table_v)
    iota = lax.iota(jnp.int32, 16)
    hvec = jnp.zeros((16,), jnp.int32) + h

    for c in range(DLOC // 16):
        k = iota + c * 16
        n = 384 - k
        an = jnp.abs(n)
        g = jnp.zeros((16,), jnp.int32)
        for t in _THRESH:
            g = g + (an >= t).astype(jnp.int32)
        bucket = jnp.where(n < 0, g + 16, g)
        dloc_v[pl.ds(c * 16, 16)] = plsc.load_gather(table_v, [bucket, hvec])

    def build_row(r, _):
        base = 128 - r0 - r
        for c in range(D2W // 16):
            idx = iota + c * 16 + base
            rows_v[pl.ds(r * D2W + c * 16, 16)] = plsc.load_gather(dloc_v, [idx])
        return 0

    lax.fori_loop(0, BQ // 2, build_row, 0, unroll=False)

    dst = pl.multiple_of((h * BQ + r0) * D2W, D2W)
    pltpu.async_copy(rows_v, d2_hbm.at[pl.ds(dst, (BQ // 2) * D2W)], sem).wait()


@jax.jit
def _sc_d2(table):
    flat = pl.kernel(
        _sc_body,
        out_type=jax.ShapeDtypeStruct((NUM_HEADS * BQ * D2W,), jnp.float32),
        mesh=plsc.VectorSubcoreMesh(core_axis_name="c", subcore_axis_name="s"),
        compiler_params=pltpu.CompilerParams(needs_layout_passes=False),
        scratch_types=[
            pltpu.VMEM((NUM_BUCKETS, NUM_HEADS), jnp.float32),
            pltpu.VMEM((DLOC,), jnp.float32),
            pltpu.VMEM(((BQ // 2) * D2W,), jnp.float32),
            pltpu.SemaphoreType.DMA,
        ],
    )(table)
    return flat.reshape(NUM_HEADS, BQ, D2W)


TBQ = 1024
NSUB = TBQ // BQ


def _tc_body(d2_ref, out_ref):
    ib = pl.program_id(1)
    i0 = ib * TBQ

    c_past = d2_ref[0, 0, 0]
    c_future = d2_ref[0, 0, D2W - 1]

    row = jax.lax.broadcasted_iota(jnp.int32, (TBQ, 1), 0)
    rgrp = row - jax.lax.rem(row, BQ)
    split = jnp.clip(i0 + rgrp - 128, 0, K - BAND) + 128
    col = jax.lax.broadcasted_iota(jnp.int32, (TBQ, K), 1)
    out_ref[0, 0] = jnp.where(col >= split, c_future, c_past)

    for sub in range(NSUB):
        i0s = i0 + sub * BQ
        cs = pl.multiple_of(jnp.clip(i0s - 128, 0, K - BAND), 128)
        ms = pl.multiple_of(cs - i0s + 256, 128)
        out_ref[0, 0, sub * BQ:(sub + 1) * BQ, pl.ds(cs, BAND)] = (
            d2_ref[0, :, pl.ds(ms, BAND)])


@jax.jit
def _tc_bias(d2):
    return pl.pallas_call(
        _tc_body,
        grid=(NUM_HEADS, Q // TBQ),
        in_specs=[pl.BlockSpec((1, BQ, D2W), lambda h, ib: (h, 0, 0))],
        out_specs=pl.BlockSpec((1, 1, TBQ, K), lambda h, ib: (0, h, ib, 0)),
        out_shape=jax.ShapeDtypeStruct((1, NUM_HEADS, Q, K), jnp.float32),
    )(d2)


def kernel(query_length, key_length, relative_attention_bias):
    del query_length, key_length
    d2 = _sc_d2(relative_attention_bias)
    return _tc_bias(d2)


# speedup vs baseline: 1.3062x; 1.0461x over previous
import jax
import jax.numpy as jnp
from jax import lax
from jax.experimental import pallas as pl
from jax.experimental.pallas import tpu as pltpu
from jax.experimental.pallas import tpu_sc as plsc

NUM_BUCKETS = 32
NUM_HEADS = 16
Q = 4096
K = 4096
BQ = 128
BAND = 384
D2W = 640
DLOC = 768

_THRESH = (1, 2, 3, 4, 5, 6, 7, 8, 12, 16, 23, 32, 46, 64, 91)


def _sc_body(table_hbm, d2_hbm, table_v, dloc_v, rows_v, sem):
    wid = lax.axis_index("s") * 2 + lax.axis_index("c")
    h = wid // 2
    r0 = (wid % 2) * (BQ // 2)

    pltpu.sync_copy(table_hbm, table_v)
    iota = lax.iota(jnp.int32, 16)
    hvec = jnp.zeros((16,), jnp.int32) + h

    for c in range(DLOC // 16):
        k = iota + c * 16
        n = 384 - k
        an = jnp.abs(n)
        g = jnp.zeros((16,), jnp.int32)
        for t in _THRESH:
            g = g + (an >= t).astype(jnp.int32)
        bucket = jnp.where(n < 0, g + 16, g)
        dloc_v[pl.ds(c * 16, 16)] = plsc.load_gather(table_v, [bucket, hvec])

    def build_row(r, _):
        base = 128 - r0 - r
        for c in range(D2W // 16):
            idx = iota + c * 16 + base
            rows_v[pl.ds(r * D2W + c * 16, 16)] = plsc.load_gather(dloc_v, [idx])
        return 0

    lax.fori_loop(0, BQ // 2, build_row, 0, unroll=False)

    dst = pl.multiple_of((h * BQ + r0) * D2W, D2W)
    pltpu.async_copy(rows_v, d2_hbm.at[pl.ds(dst, (BQ // 2) * D2W)], sem).wait()


@jax.jit
def _sc_d2(table):
    flat = pl.kernel(
        _sc_body,
        out_type=jax.ShapeDtypeStruct((NUM_HEADS * BQ * D2W,), jnp.float32),
        mesh=plsc.VectorSubcoreMesh(core_axis_name="c", subcore_axis_name="s"),
        compiler_params=pltpu.CompilerParams(needs_layout_passes=False),
        scratch_types=[
            pltpu.VMEM((NUM_BUCKETS, NUM_HEADS), jnp.float32),
            pltpu.VMEM((DLOC,), jnp.float32),
            pltpu.VMEM(((BQ // 2) * D2W,), jnp.float32),
            pltpu.SemaphoreType.DMA,
        ],
    )(table)
    return flat.reshape(NUM_HEADS, BQ, D2W)


TBQ = 1024
NSUB = TBQ // BQ


def _fill_and_bands(out_ref, ib, c_past, c_future, band_of):
    i0 = ib * TBQ
    row = jax.lax.broadcasted_iota(jnp.int32, (TBQ, 1), 0)
    rgrp = row - jax.lax.rem(row, BQ)
    split = jnp.clip(i0 + rgrp - 128, 0, K - BAND) + 128
    col = jax.lax.broadcasted_iota(jnp.int32, (TBQ, K), 1)
    out_ref[0, 0] = jnp.where(col >= split, c_future, c_past)

    for sub in range(NSUB):
        i0s = i0 + sub * BQ
        cs = pl.multiple_of(jnp.clip(i0s - 128, 0, K - BAND), 128)
        ms = pl.multiple_of(cs - i0s + 256, 128)
        out_ref[0, 0, sub * BQ:(sub + 1) * BQ, pl.ds(cs, BAND)] = band_of(ms)


def _tc0_body(tt_ref, out_ref, d2s_ref):
    ib = pl.program_id(0)

    @pl.when(ib == 0)
    def _build_d2():
        r = jax.lax.broadcasted_iota(jnp.int32, (BQ, D2W), 0)
        m = jax.lax.broadcasted_iota(jnp.int32, (BQ, D2W), 1)
        n = r + 256 - m
        an = jnp.abs(n)
        g = jnp.zeros((BQ, D2W), jnp.int32)
        for t in _THRESH:
            g = g + (an >= t).astype(jnp.int32)
        bucket = jnp.where(n < 0, g + 16, g)
        acc = jnp.zeros((BQ, D2W), jnp.float32)
        for b in range(NUM_BUCKETS):
            acc = jnp.where(bucket == b, tt_ref[0, 0, b], acc)
        d2s_ref[...] = acc

    _fill_and_bands(out_ref, ib, tt_ref[0, 0, 15], tt_ref[0, 0, 31],
                    lambda ms: d2s_ref[:, pl.ds(ms, BAND)])


def _tcr_body(d2_ref, prev_ref, out_ref):
    del prev_ref
    ib = pl.program_id(1)
    _fill_and_bands(out_ref, ib, d2_ref[0, 0, 0], d2_ref[0, 0, D2W - 1],
                    lambda ms: d2_ref[0, :, pl.ds(ms, BAND)])


@jax.jit
def _tc_head0(tt3):
    return pl.pallas_call(
        _tc0_body,
        grid=(Q // TBQ,),
        in_specs=[pl.BlockSpec((1, 1, NUM_BUCKETS), lambda ib: (0, 0, 0))],
        out_specs=pl.BlockSpec((1, 1, TBQ, K), lambda ib: (0, 0, ib, 0)),
        out_shape=jax.ShapeDtypeStruct((1, NUM_HEADS, Q, K), jnp.float32),
        scratch_shapes=[pltpu.VMEM((BQ, D2W), jnp.float32)],
    )(tt3)


@jax.jit
def _tc_rest(d2, part):
    return pl.pallas_call(
        _tcr_body,
        grid=(NUM_HEADS - 1, Q // TBQ),
        in_specs=[
            pl.BlockSpec((1, BQ, D2W), lambda h, ib: (h + 1, 0, 0)),
            pl.BlockSpec(memory_space=pltpu.HBM),
        ],
        out_specs=pl.BlockSpec((1, 1, TBQ, K), lambda h, ib: (0, h + 1, ib, 0)),
        out_shape=jax.ShapeDtypeStruct((1, NUM_HEADS, Q, K), jnp.float32),
        input_output_aliases={1: 0},
    )(d2, part)


def kernel(query_length, key_length, relative_attention_bias):
    del query_length, key_length
    tt3 = relative_attention_bias.T.reshape(NUM_HEADS, 1, NUM_BUCKETS)
    part = _tc_head0(tt3)
    d2 = _sc_d2(relative_attention_bias)
    return _tc_rest(d2, part)
